# baseline (speedup 1.0000x reference)
import jax
import jax.numpy as jnp
from jax import lax
from jax.experimental import pallas as pl
from jax.experimental.pallas import tpu as pltpu
from jax.experimental.pallas import tpu_sc as plsc

N = 10000
E = 320000
D = 128

NC = 2
NS = 16
NW = NC * NS
LANES = 16
EPT = E // NW
C = 80
NCHUNK = EPT // C
RPT = N // NS
FR = 125

_MESH = plsc.VectorSubcoreMesh(core_axis_name="c", subcore_axis_name="s")
_SC_PARAMS = pltpu.CompilerParams(needs_layout_passes=False,
                                  use_tc_tiling_on_sc=False)


def _worker_id():
    return lax.axis_index("s") * NC + lax.axis_index("c")


def _deg_body(dst_hbm, ew_hbm, out_hbm, deg_v, dst_a, ew_a):
    wid = _worker_id()
    pltpu.sync_copy(dst_hbm.at[wid], dst_a)
    pltpu.sync_copy(ew_hbm.at[wid], ew_a)
    zero16 = jnp.zeros((LANES,), jnp.float32)

    def _zero(i, carry):
        deg_v[pl.ds(i * LANES, LANES)] = zero16
        return carry

    lax.fori_loop(0, N // LANES, _zero, 0)

    def _grp(j, carry):
        idx = dst_a[pl.ds(j * LANES, LANES)]
        w = ew_a[pl.ds(j * LANES, LANES)]
        plsc.addupdate_scatter(deg_v, [idx], w)
        return carry

    lax.fori_loop(0, EPT // LANES, _grp, 0)
    pltpu.sync_copy(deg_v, out_hbm.at[wid])


_deg_call = pl.kernel(
    _deg_body,
    out_type=jax.ShapeDtypeStruct((NW, N), jnp.float32),
    mesh=_MESH,
    compiler_params=_SC_PARAMS,
    scratch_types=[
        pltpu.VMEM((N,), jnp.float32),
        pltpu.VMEM((EPT,), jnp.int32),
        pltpu.VMEM((EPT,), jnp.float32),
    ],
)


NGRP = C // LANES


def _agg_body(g_hbm, pk_hbm, ew_hbm, out_hbm,
              acc, pk_a, ew_a, gb0, gb1, sb0, sb1,
              semg0, semg1, sems0, sems1):
    cid = lax.axis_index("c")
    sid = lax.axis_index("s")
    wid = sid * NC + cid
    zero16 = jnp.zeros((LANES,), jnp.float32)
    zidx16 = jnp.zeros((LANES,), jnp.int32)

    pltpu.sync_copy(pk_hbm.at[wid], pk_a)
    pltpu.sync_copy(ew_hbm.at[wid], ew_a)

    def _zr(r, carry):
        for cb in range(D // LANES):
            sb0[r, pl.ds(cb * LANES, LANES)] = zero16
        return carry

    lax.fori_loop(0, C, _zr, 0)

    def _za(k, carry):
        pltpu.sync_copy(sb0, acc.at[pl.ds(sid * RPT + k * C, C)])
        return carry

    lax.fori_loop(0, RPT // C, _za, 0)
    pltpu.sync_copy(sb0.at[pl.ds(0, RPT % C)],
                    acc.at[pl.ds(sid * RPT + (RPT // C) * C, RPT % C)])
    plsc.subcore_barrier()

    gbufs = (gb0, gb1)
    sbufs = (sb0, sb1)
    semg = (semg0, semg1)
    sems = (sems0, sems1)

    def _issue_gathers(j, bq, sg):
        jj = jnp.where(j < NCHUNK, j, 0)

        def _g(g, carry):
            pk16 = pk_a[pl.ds(jj * C + g * LANES, LANES)]
            s16 = jnp.bitwise_and(pk16, 0xFFFF)
            pltpu.async_copy(g_hbm.at[s16], bq.at[pl.ds(g * LANES, LANES)],
                             sg)
            return carry

        lax.fori_loop(0, NGRP, _g, 0)

    def _wait_gathers(bq, sg):
        for _ in range(NGRP):
            pltpu.make_async_copy(g_hbm.at[zidx16],
                                  bq.at[pl.ds(0, LANES)], sg).wait()

    def _scale(j, bp, sp):
        def _g(g, carry):
            base = j * C + g * LANES
            w16 = ew_a[pl.ds(base, LANES)]
            e0 = g * LANES
            for k in range(LANES):
                w = w16.at[jnp.full((LANES,), k, jnp.int32)].get(
                    mode="promise_in_bounds")
                for cb in range(D // 32):
                    x32 = bp[e0 + k, pl.ds(cb * 32, 32)]
                    lo, hi = plsc.unpack(
                        x32, format=plsc.PackFormat.INTERLEAVED,
                        preferred_element_type=jnp.float32)
                    sp[e0 + k, pl.ds(cb * 32, LANES)] = lo * w
                    sp[e0 + k, pl.ds(cb * 32 + LANES, LANES)] = hi * w
            return carry

        lax.fori_loop(0, NGRP, _g, 0)

    def _issue_scatters(j, sp, ss):
        def _g(g, carry):
            pk16 = pk_a[pl.ds(j * C + g * LANES, LANES)]
            d16 = lax.shift_right_logical(pk16, 16)
            pltpu.async_copy(sp.at[pl.ds(g * LANES, LANES)], acc.at[d16],
                             ss, add=True)
            return carry

        lax.fori_loop(0, NGRP, _g, 0)

    def _wait_scatters(sp, ss):
        for _ in range(NGRP):
            pltpu.make_async_copy(sp.at[pl.ds(0, LANES)], acc.at[zidx16],
                                  ss).wait()

    def _step(j, p, skip_scatter_wait=False):
        q = 1 - p
        _wait_gathers(gbufs[p], semg[p])
        _issue_gathers(j + 1, gbufs[q], semg[q])
        if not skip_scatter_wait:
            _wait_scatters(sbufs[p], sems[p])
        _scale(j, gbufs[p], sbufs[p])
        _issue_scatters(j, sbufs[p], sems[p])

    _issue_gathers(0, gb0, semg0)
    _step(0, 0, skip_scatter_wait=True)
    _step(1, 1, skip_scatter_wait=True)
    _step(2, 0)

    def _duo(t, carry):
        j = 3 + 2 * t
        _step(j, 1)
        _step(j + 1, 0)
        return carry

    lax.fori_loop(0, (NCHUNK - 3) // 2, _duo, 0)
    _wait_scatters(sb1, sems1)
    _wait_scatters(sb0, sems0)
    _wait_gathers(gb1, semg1)

    plsc.subcore_barrier()

    def _flush(k, carry):
        r0 = sid * RPT + k * C
        pltpu.sync_copy(acc.at[pl.ds(r0, C)], sb0)
        pltpu.sync_copy(sb0, out_hbm.at[cid, pl.ds(r0, C)])
        return carry

    lax.fori_loop(0, RPT // C, _flush, 0)
    rtail = RPT % C
    r0t = sid * RPT + (RPT // C) * C
    pltpu.sync_copy(acc.at[pl.ds(r0t, rtail)], sb0.at[pl.ds(0, rtail)])
    pltpu.sync_copy(sb0.at[pl.ds(0, rtail)],
                    out_hbm.at[cid, pl.ds(r0t, rtail)])


_agg_call = pl.kernel(
    _agg_body,
    out_type=jax.ShapeDtypeStruct((NC, N, D), jnp.float32),
    mesh=_MESH,
    compiler_params=_SC_PARAMS,
    scratch_types=[
        pltpu.VMEM_SHARED((N, D), jnp.float32),
        pltpu.VMEM((EPT,), jnp.int32),
        pltpu.VMEM((EPT,), jnp.float32),
        pltpu.VMEM((C, D), jnp.bfloat16),
        pltpu.VMEM((C, D), jnp.bfloat16),
        pltpu.VMEM((C, D), jnp.float32),
        pltpu.VMEM((C, D), jnp.float32),
        pltpu.SemaphoreType.DMA,
        pltpu.SemaphoreType.DMA,
        pltpu.SemaphoreType.DMA,
        pltpu.SemaphoreType.DMA,
    ],
)


def _tc1_body(x_ref, w1_ref, degp_ref, src_ref, dst_ref,
              g1_ref, dis_ref, pk_ref):
    deg = jnp.sum(degp_ref[...], axis=0) + 1.0
    dis = jnp.where(deg > 0, lax.rsqrt(deg), 0.0)
    h = jnp.dot(x_ref[...], w1_ref[...], preferred_element_type=jnp.float32)
    g1_ref[...] = h * dis[:, None]
    dis_ref[...] = dis
    pk_ref[...] = jnp.bitwise_or(src_ref[...],
                                 lax.shift_left(dst_ref[...], 16))


def _tc1(x, W1, degp, src, dst):
    return pl.pallas_call(
        _tc1_body,
        out_shape=(
            jax.ShapeDtypeStruct((N, D), jnp.float32),
            jax.ShapeDtypeStruct((N,), jnp.float32),
            jax.ShapeDtypeStruct((E,), jnp.int32),
        ),
    )(x, W1, degp, src, dst)


def _tc2_body(p_ref, g1_ref, dis_ref, b1_ref, gnw_ref, gnb_ref, gnms_ref,
              w2_ref, g2_ref):
    dis = dis_ref[...]
    out1 = (p_ref[0] + p_ref[1] + g1_ref[...]) * dis[:, None] + b1_ref[...][None, :]
    mean = jnp.mean(out1, axis=0, keepdims=True)
    xc = out1 - gnms_ref[...][None, :] * mean
    var = jnp.mean(xc * xc, axis=0, keepdims=True)
    y = gnw_ref[...][None, :] * xc / jnp.sqrt(var + 1e-5) + gnb_ref[...][None, :]
    h2 = jnp.maximum(y, 0.0)
    hw = jnp.dot(h2, w2_ref[...], preferred_element_type=jnp.float32)
    g2_ref[...] = hw * dis[:, None]


def _tc2(p, g1, dis, b1, gn_weight, gn_bias, gn_mean_scale, W2):
    return pl.pallas_call(
        _tc2_body,
        out_shape=jax.ShapeDtypeStruct((N, D), jnp.float32),
    )(p, g1, dis, b1, gn_weight, gn_bias, gn_mean_scale, W2)


def _tc3_body(p_ref, g2_ref, dis_ref, b2_ref, out_ref):
    out_ref[...] = ((p_ref[0] + p_ref[1] + g2_ref[...])
                    * dis_ref[...][:, None] + b2_ref[...][None, :])


def _tc3(p, g2, dis, b2):
    return pl.pallas_call(
        _tc3_body,
        out_shape=jax.ShapeDtypeStruct((N, D), jnp.float32),
    )(p, g2, dis, b2)


def _ileave(g):
    gb = g.astype(jnp.bfloat16)
    return gb.reshape(N, D // 32, 2, 16).swapaxes(2, 3).reshape(N, D)


def kernel(x, edge_index, edge_weight, W1, b1, gn_weight, gn_bias,
           gn_mean_scale, W2, b2):
    src = edge_index[0]
    dst = edge_index[1]
    ew_r = edge_weight.reshape(NW, EPT)
    degp = _deg_call(dst.reshape(NW, EPT), ew_r)
    g1, dis, pk = _tc1(x, W1, degp, src, dst)
    pk_r = pk.reshape(NW, EPT)
    p1 = _agg_call(_ileave(g1), pk_r, ew_r)
    g2 = _tc2(p1, g1, dis, b1, gn_weight, gn_bias, gn_mean_scale, W2)
    p2 = _agg_call(_ileave(g2), pk_r, ew_r)
    out = _tc3(p2, g2, dis, b2)
    return out


# speedup vs baseline: 1.9785x; 1.9785x over previous
import jax
import jax.numpy as jnp
from jax import lax
from jax.experimental import pallas as pl
from jax.experimental.pallas import tpu as pltpu
from jax.experimental.pallas import tpu_sc as plsc

N = 10000
E = 320000
D = 128

NC = 2
NS = 16
NW = NC * NS
LANES = 16
EPT = E // NW
C = 80
NCHUNK = EPT // C
RPT = N // NS
FR = 125

_MESH = plsc.VectorSubcoreMesh(core_axis_name="c", subcore_axis_name="s")
_SC_PARAMS = pltpu.CompilerParams(needs_layout_passes=False,
                                  use_tc_tiling_on_sc=False)


def _worker_id():
    return lax.axis_index("s") * NC + lax.axis_index("c")


def _deg_body(dst_hbm, ew_hbm, out_hbm, deg_v, dst_a, ew_a):
    wid = _worker_id()
    pltpu.sync_copy(dst_hbm.at[wid], dst_a)
    pltpu.sync_copy(ew_hbm.at[wid], ew_a)
    zero16 = jnp.zeros((LANES,), jnp.float32)

    def _zero(i, carry):
        deg_v[pl.ds(i * LANES, LANES)] = zero16
        return carry

    lax.fori_loop(0, N // LANES, _zero, 0)

    def _grp(j, carry):
        idx = dst_a[pl.ds(j * LANES, LANES)]
        w = ew_a[pl.ds(j * LANES, LANES)]
        plsc.addupdate_scatter(deg_v, [idx], w)
        return carry

    lax.fori_loop(0, EPT // LANES, _grp, 0)
    pltpu.sync_copy(deg_v, out_hbm.at[wid])


_deg_call = pl.kernel(
    _deg_body,
    out_type=jax.ShapeDtypeStruct((NW, N), jnp.float32),
    mesh=_MESH,
    compiler_params=_SC_PARAMS,
    scratch_types=[
        pltpu.VMEM((N,), jnp.float32),
        pltpu.VMEM((EPT,), jnp.int32),
        pltpu.VMEM((EPT,), jnp.float32),
    ],
)


NGRP = C // LANES


def _agg_body(g_hbm, pk_hbm, ew_hbm, out_hbm,
              acc, pk_a, ew_a, b0, b1, b2,
              semg0, semg1, semg2, sems0, sems1, sems2):
    cid = lax.axis_index("c")
    sid = lax.axis_index("s")
    wid = sid * NC + cid
    zero16 = jnp.zeros((LANES,), jnp.float32)
    zidx16 = jnp.zeros((LANES,), jnp.int32)

    pltpu.sync_copy(pk_hbm.at[wid], pk_a)
    pltpu.sync_copy(ew_hbm.at[wid], ew_a)

    def _zr(r, carry):
        for cb in range(D // LANES):
            b0[r, pl.ds(cb * LANES, LANES)] = zero16
        return carry

    lax.fori_loop(0, C, _zr, 0)

    def _za(k, carry):
        pltpu.sync_copy(b0, acc.at[pl.ds(sid * RPT + k * C, C)])
        return carry

    lax.fori_loop(0, RPT // C, _za, 0)
    pltpu.sync_copy(b0.at[pl.ds(0, RPT % C)],
                    acc.at[pl.ds(sid * RPT + (RPT // C) * C, RPT % C)])
    plsc.subcore_barrier()

    def _issue_gathers(j, bq, semg):
        jj = jnp.where(j < NCHUNK, j, 0)

        def _g(g, carry):
            pk16 = pk_a[pl.ds(jj * C + g * LANES, LANES)]
            s16 = jnp.bitwise_and(pk16, 0xFFFF)
            pltpu.async_copy(g_hbm.at[s16], bq.at[pl.ds(g * LANES, LANES)],
                             semg)
            return carry

        lax.fori_loop(0, NGRP, _g, 0)

    def _wait_gathers(bq, semg):
        for _ in range(NGRP):
            pltpu.make_async_copy(g_hbm.at[zidx16],
                                  bq.at[pl.ds(0, LANES)], semg).wait()

    def _scale(j, bp):
        def _g(g, carry):
            base = j * C + g * LANES
            w16 = ew_a[pl.ds(base, LANES)]
            e0 = g * LANES
            for k in range(LANES):
                w = w16.at[jnp.full((LANES,), k, jnp.int32)].get(
                    mode="promise_in_bounds")
                for cb in range(D // LANES):
                    sl = pl.ds(cb * LANES, LANES)
                    bp[e0 + k, sl] = bp[e0 + k, sl] * w
            return carry

        lax.fori_loop(0, NGRP, _g, 0)

    def _issue_scatters(j, bp, sems):
        def _g(g, carry):
            pk16 = pk_a[pl.ds(j * C + g * LANES, LANES)]
            d16 = lax.shift_right_logical(pk16, 16)
            pltpu.async_copy(bp.at[pl.ds(g * LANES, LANES)], acc.at[d16],
                             sems, add=True)
            return carry

        lax.fori_loop(0, NGRP, _g, 0)

    def _wait_scatters(bp, sems):
        for _ in range(NGRP):
            pltpu.make_async_copy(bp.at[pl.ds(0, LANES)], acc.at[zidx16],
                                  sems).wait()

    bufs = (b0, b1, b2)
    semg = (semg0, semg1, semg2)
    sems = (sems0, sems1, sems2)

    def _step(j, p, q, skip_scatter_wait=False):
        _wait_gathers(bufs[p], semg[p])
        _scale(j, bufs[p])
        _issue_scatters(j, bufs[p], sems[p])
        if not skip_scatter_wait:
            _wait_scatters(bufs[q], sems[q])
        _issue_gathers(j + 2, bufs[q], semg[q])

    _issue_gathers(0, b0, semg0)
    _issue_gathers(1, b1, semg1)
    _step(0, 0, 2, skip_scatter_wait=True)
    _step(1, 1, 0)

    def _trip(t, carry):
        j = 2 + 3 * t
        _step(j, 2, 1)
        _step(j + 1, 0, 2)
        _step(j + 2, 1, 0)
        return carry

    lax.fori_loop(0, (NCHUNK - 2) // 3, _trip, 0)
    _wait_scatters(b1, sems1)
    _wait_gathers(b2, semg2)
    _wait_gathers(b0, semg0)

    plsc.subcore_barrier()

    def _flush(k, carry):
        r0 = sid * RPT + k * C
        pltpu.sync_copy(acc.at[pl.ds(r0, C)], b0)
        pltpu.sync_copy(b0, out_hbm.at[cid, pl.ds(r0, C)])
        return carry

    lax.fori_loop(0, RPT // C, _flush, 0)
    rtail = RPT % C
    r0t = sid * RPT + (RPT // C) * C
    pltpu.sync_copy(acc.at[pl.ds(r0t, rtail)], b0.at[pl.ds(0, rtail)])
    pltpu.sync_copy(b0.at[pl.ds(0, rtail)], out_hbm.at[cid, pl.ds(r0t, rtail)])


_agg_call = pl.kernel(
    _agg_body,
    out_type=jax.ShapeDtypeStruct((NC, N, D), jnp.float32),
    mesh=_MESH,
    compiler_params=_SC_PARAMS,
    scratch_types=[
        pltpu.VMEM_SHARED((N, D), jnp.float32),
        pltpu.VMEM((EPT,), jnp.int32),
        pltpu.VMEM((EPT,), jnp.float32),
        pltpu.VMEM((C, D), jnp.float32),
        pltpu.VMEM((C, D), jnp.float32),
        pltpu.VMEM((C, D), jnp.float32),
        pltpu.SemaphoreType.DMA,
        pltpu.SemaphoreType.DMA,
        pltpu.SemaphoreType.DMA,
        pltpu.SemaphoreType.DMA,
        pltpu.SemaphoreType.DMA,
        pltpu.SemaphoreType.DMA,
    ],
)


def _tc1_body(x_ref, w1_ref, degp_ref, src_ref, dst_ref,
              g1_ref, dis_ref, pk_ref):
    deg = jnp.sum(degp_ref[...], axis=0) + 1.0
    dis = jnp.where(deg > 0, lax.rsqrt(deg), 0.0)
    h = jnp.dot(x_ref[...], w1_ref[...], preferred_element_type=jnp.float32)
    g1_ref[...] = h * dis[:, None]
    dis_ref[...] = dis
    pk_ref[...] = jnp.bitwise_or(src_ref[...],
                                 lax.shift_left(dst_ref[...], 16))


def _tc1(x, W1, degp, src, dst):
    return pl.pallas_call(
        _tc1_body,
        out_shape=(
            jax.ShapeDtypeStruct((N, D), jnp.float32),
            jax.ShapeDtypeStruct((N,), jnp.float32),
            jax.ShapeDtypeStruct((E,), jnp.int32),
        ),
    )(x, W1, degp, src, dst)


def _tc2_body(p_ref, g1_ref, dis_ref, b1_ref, gnw_ref, gnb_ref, gnms_ref,
              w2_ref, g2_ref):
    dis = dis_ref[...]
    out1 = (p_ref[0] + p_ref[1] + g1_ref[...]) * dis[:, None] + b1_ref[...][None, :]
    mean = jnp.mean(out1, axis=0, keepdims=True)
    xc = out1 - gnms_ref[...][None, :] * mean
    var = jnp.mean(xc * xc, axis=0, keepdims=True)
    y = gnw_ref[...][None, :] * xc / jnp.sqrt(var + 1e-5) + gnb_ref[...][None, :]
    h2 = jnp.maximum(y, 0.0)
    hw = jnp.dot(h2, w2_ref[...], preferred_element_type=jnp.float32)
    g2_ref[...] = hw * dis[:, None]


def _tc2(p, g1, dis, b1, gn_weight, gn_bias, gn_mean_scale, W2):
    return pl.pallas_call(
        _tc2_body,
        out_shape=jax.ShapeDtypeStruct((N, D), jnp.float32),
    )(p, g1, dis, b1, gn_weight, gn_bias, gn_mean_scale, W2)


def _tc3_body(p_ref, g2_ref, dis_ref, b2_ref, out_ref):
    out_ref[...] = ((p_ref[0] + p_ref[1] + g2_ref[...])
                    * dis_ref[...][:, None] + b2_ref[...][None, :])


def _tc3(p, g2, dis, b2):
    return pl.pallas_call(
        _tc3_body,
        out_shape=jax.ShapeDtypeStruct((N, D), jnp.float32),
    )(p, g2, dis, b2)


def kernel(x, edge_index, edge_weight, W1, b1, gn_weight, gn_bias,
           gn_mean_scale, W2, b2):
    src = edge_index[0]
    dst = edge_index[1]
    ew_r = edge_weight.reshape(NW, EPT)
    degp = _deg_call(dst.reshape(NW, EPT), ew_r)
    g1, dis, pk = _tc1(x, W1, degp, src, dst)
    pk_r = pk.reshape(NW, EPT)
    p1 = _agg_call(g1, pk_r, ew_r)
    g2 = _tc2(p1, g1, dis, b1, gn_weight, gn_bias, gn_mean_scale, W2)
    p2 = _agg_call(g2, pk_r, ew_r)
    out = _tc3(p2, g2, dis, b2)
    return out
